# baseline (speedup 1.0000x reference)
import functools

import jax
import jax.numpy as jnp
from jax import lax
from jax.experimental import pallas as pl
from jax.experimental.pallas import tpu as pltpu
from jax.experimental.pallas import tpu_sc as plsc

MASK_IDX = 32
ZERO_IDX = 33
D = 1024
LANES = 16
NUM_WORKERS = 32
ROWS_PER_BLOCK = 32
NBUF = 2
KUNROLL = 4
TABLE_ROWS = 34


@functools.lru_cache(maxsize=None)
def _build_sc_kernel(n_tokens: int):
    per_w = n_tokens // NUM_WORKERS
    n_sub = per_w // ROWS_PER_BLOCK
    mesh = plsc.VectorSubcoreMesh(core_axis_name="c", subcore_axis_name="s")

    @functools.partial(
        pl.kernel,
        mesh=mesh,
        out_type=jax.ShapeDtypeStruct((n_tokens, D), jnp.float32),
        compiler_params=pltpu.CompilerParams(needs_layout_passes=False),
        scratch_types=[
            pltpu.VMEM((per_w,), jnp.int32),
            pltpu.VMEM((per_w,), jnp.int32),
            pltpu.VMEM((per_w,), jnp.int32),
            pltpu.VMEM((per_w,), jnp.int32),
            pltpu.VMEM((TABLE_ROWS, D), jnp.float32),
            pltpu.VMEM((NBUF, ROWS_PER_BLOCK, D), jnp.float32),
            pltpu.SemaphoreType.DMA,
            pltpu.SemaphoreType.DMA,
        ],
    )
    def sc_embed(table_hbm, tok_hbm, aa_hbm, pad_hbm, out_hbm,
                 tok_v, aa_v, pad_v, idx_v, table_v, bufs, sem0, sem1):
        wid = lax.axis_index("s") * 2 + lax.axis_index("c")
        base = wid * per_w

        pltpu.sync_copy(table_hbm, table_v)
        pltpu.sync_copy(tok_hbm.at[pl.ds(base, per_w)], tok_v)
        pltpu.sync_copy(aa_hbm.at[pl.ds(base, per_w)], aa_v)
        pltpu.sync_copy(pad_hbm.at[pl.ds(base, per_w)], pad_v)

        def idx_body(i, carry):
            sl = pl.ds(pl.multiple_of(i * LANES, LANES), LANES)
            t = tok_v[sl]
            a = aa_v[sl]
            p = pad_v[sl]
            idx = jnp.where(a != 0, MASK_IDX, t)
            idx_v[sl] = jnp.where(p != 0, ZERO_IDX, idx)
            return carry

        lax.fori_loop(0, per_w // LANES, idx_body, 0)

        sems = (sem0, sem1)

        def fill_block(j, b):
            buf = bufs.at[b]

            def grp_body(g, carry):
                sl16 = pl.ds(pl.multiple_of(
                    j * ROWS_PER_BLOCK + g * LANES, LANES), LANES)
                idxg = idx_v[sl16]
                rows = [idxg[l] for l in range(LANES)]

                def k_body(k, carry2):
                    for u in range(KUNROLL):
                        sl = pl.ds(pl.multiple_of(
                            (k * KUNROLL + u) * LANES, LANES), LANES)
                        ws = [table_v[rows[l], sl] for l in range(LANES)]
                        for l in range(LANES):
                            buf[g * LANES + l, sl] = ws[l]
                    return carry2

                lax.fori_loop(0, D // LANES // KUNROLL, k_body, 0)
                return carry

            lax.fori_loop(0, ROWS_PER_BLOCK // LANES, grp_body, 0)

        def out_desc(j, b):
            off = pl.multiple_of(j * ROWS_PER_BLOCK, ROWS_PER_BLOCK)
            return pltpu.make_async_copy(
                bufs.at[b], out_hbm.at[pl.ds(base + off, ROWS_PER_BLOCK)],
                sems[b])

        for b in range(NBUF):
            fill_block(b, b)
            out_desc(b, b).start()

        def ring_body(jj, carry):
            for b in range(NBUF):
                j = jj * NBUF + b

                @pl.when(j < n_sub - NBUF)
                def _():
                    out_desc(j, b).wait()
                    fill_block(j + NBUF, b)
                    out_desc(j + NBUF, b).start()
            return carry

        lax.fori_loop(0, n_sub // NBUF, ring_body, 0)
        for b in range(NBUF):
            out_desc(n_sub - NBUF + b, b).wait()

    return sc_embed


def kernel(tokens, padding_mask, mask_aa, table):
    B, L = tokens.shape
    tok = tokens.reshape(-1).astype(jnp.int32)
    aa = mask_aa.reshape(-1).astype(jnp.int32)
    pad = padding_mask.reshape(-1).astype(jnp.int32)
    table_padded = jnp.concatenate(
        [table, jnp.zeros((1, table.shape[1]), table.dtype)], axis=0)
    out = _build_sc_kernel(B * L)(table_padded, tok, aa, pad)
    return out.reshape(B, L, D)


# speedup vs baseline: 1.7093x; 1.7093x over previous
import functools

import jax
import jax.numpy as jnp
from jax import lax
from jax.experimental import pallas as pl
from jax.experimental.pallas import tpu as pltpu
from jax.experimental.pallas import tpu_sc as plsc

MASK_IDX = 32
ZERO_IDX = 33
D = 1024
LANES = 16
NUM_WORKERS = 32
WINDOW_GROUPS = 2
TABLE_ROWS = 34


@functools.lru_cache(maxsize=None)
def _build_sc_kernel(n_tokens: int):
    per_w = n_tokens // NUM_WORKERS
    n_grp = per_w // LANES
    mesh = plsc.VectorSubcoreMesh(core_axis_name="c", subcore_axis_name="s")

    @functools.partial(
        pl.kernel,
        mesh=mesh,
        out_type=jax.ShapeDtypeStruct((n_tokens, D), jnp.float32),
        compiler_params=pltpu.CompilerParams(needs_layout_passes=False),
        scratch_types=[
            pltpu.VMEM((per_w,), jnp.int32),
            pltpu.VMEM((per_w,), jnp.int32),
            pltpu.VMEM((per_w,), jnp.int32),
            pltpu.VMEM((per_w,), jnp.int32),
            pltpu.VMEM((TABLE_ROWS, D), jnp.float32),
            pltpu.SemaphoreType.DMA,
        ],
    )
    def sc_embed(table_hbm, tok_hbm, aa_hbm, pad_hbm, out_hbm,
                 tok_v, aa_v, pad_v, idx_v, table_v, sem):
        wid = lax.axis_index("s") * 2 + lax.axis_index("c")
        base = wid * per_w

        pltpu.sync_copy(table_hbm, table_v)
        pltpu.sync_copy(tok_hbm.at[pl.ds(base, per_w)], tok_v)
        pltpu.sync_copy(aa_hbm.at[pl.ds(base, per_w)], aa_v)
        pltpu.sync_copy(pad_hbm.at[pl.ds(base, per_w)], pad_v)

        def idx_body(i, carry):
            sl = pl.ds(pl.multiple_of(i * LANES, LANES), LANES)
            t = tok_v[sl]
            a = aa_v[sl]
            p = pad_v[sl]
            idx = jnp.where(a != 0, MASK_IDX, t)
            idx_v[sl] = jnp.where(p != 0, ZERO_IDX, idx)
            return carry

        lax.fori_loop(0, per_w // LANES, idx_body, 0)

        def drain_group():
            pltpu.make_async_copy(
                table_hbm.at[pl.ds(0, LANES)],
                out_hbm.at[pl.ds(base, LANES)], sem).wait()

        def grp_body(g, carry):
            sl16 = pl.ds(pl.multiple_of(g * LANES, LANES), LANES)
            idxg = idx_v[sl16]
            for l in range(LANES):
                pltpu.async_copy(
                    table_v.at[idxg[l]],
                    out_hbm.at[base + g * LANES + l], sem)

            @pl.when(g >= WINDOW_GROUPS)
            def _():
                drain_group()
            return carry

        lax.fori_loop(0, n_grp, grp_body, 0)
        for _ in range(WINDOW_GROUPS):
            drain_group()

    return sc_embed


def kernel(tokens, padding_mask, mask_aa, table):
    B, L = tokens.shape
    tok = tokens.reshape(-1).astype(jnp.int32)
    aa = mask_aa.reshape(-1).astype(jnp.int32)
    pad = padding_mask.reshape(-1).astype(jnp.int32)
    table_padded = jnp.concatenate(
        [table, jnp.zeros((1, table.shape[1]), table.dtype)], axis=0)
    out = _build_sc_kernel(B * L)(table_padded, tok, aa, pad)
    return out.reshape(B, L, D)
